# baseline (speedup 1.0000x reference)
import functools

import jax
import jax.numpy as jnp
from jax import lax
from jax.experimental import pallas as pl
from jax.experimental.pallas import tpu as pltpu
from jax.experimental.pallas import tpu_sc as plsc

_C = 32
_NUM_CORES = 2
_NUM_SUBCORES = 16
_LANES = 16
_CORE0_FRAC = 0.84


def _even_ceil(x):
    v = int(-(-x // 1))
    return v + (v % 2)


def _make_sc_kernel(cpw0, cpw1, n_streams, d_feat, b_last):
    n_chunks = _NUM_SUBCORES * (cpw0 + cpw1)
    b_pad = n_chunks * _C
    inv_s = 1.0 / float(n_streams - 1)
    n_groups = d_feat // _LANES
    chunk_words = n_streams * _C
    mesh = plsc.VectorSubcoreMesh(
        core_axis_name="core", subcore_axis_name="subcore",
        num_cores=_NUM_CORES)

    @functools.partial(
        pl.kernel,
        out_type=jax.ShapeDtypeStruct((b_pad, d_feat), jnp.float32),
        mesh=mesh,
        scratch_types=[
            pltpu.VMEM((2 * chunk_words,), jnp.int32),
            pltpu.VMEM((2, chunk_words, d_feat), jnp.float32),
            pltpu.VMEM((2, _C, d_feat), jnp.float32),
            pltpu.SemaphoreType.DMA,
            pltpu.SemaphoreType.DMA,
            pltpu.SemaphoreType.DMA,
            pltpu.SemaphoreType.DMA,
            pltpu.SemaphoreType.DMA,
            pltpu.SemaphoreType.DMA,
        ],
    )
    def sc_kernel(idx_hbm, table_hbm, out_hbm, idx_v, rows_v, out_v,
                  i_sem0, i_sem1, g_sem0, g_sem1, o_sem0, o_sem1):
        core = lax.axis_index("core")
        sub = lax.axis_index("subcore")
        my_cpw = jnp.where(core == 0, cpw0, cpw1)
        c0 = jnp.where(core == 0, sub * cpw0,
                       _NUM_SUBCORES * cpw0 + sub * cpw1)
        i_sems = (i_sem0, i_sem1)
        g_sems = (g_sem0, g_sem1)
        o_sems = (o_sem0, o_sem1)

        def idx_copy(j, buf):
            return pltpu.make_async_copy(
                idx_hbm.at[pl.ds((c0 + j) * chunk_words, chunk_words)],
                idx_v.at[pl.ds(buf * chunk_words, chunk_words)], i_sems[buf])

        def gather_copy(buf):
            return pltpu.make_async_copy(
                table_hbm.at[idx_v.at[pl.ds(buf * chunk_words, chunk_words)]],
                rows_v.at[buf], g_sems[buf])

        def issue_gathers(buf):
            gather_copy(buf).start()

        def drain_gathers(buf):
            gather_copy(buf).wait()

        def out_copy(j, buf):
            return pltpu.make_async_copy(
                out_v.at[buf], out_hbm.at[pl.ds((c0 + j) * _C, _C)],
                o_sems[buf])

        def compute(j, buf):
            row0 = (c0 + j) * _C

            @plsc.parallel_loop(0, _C, 1, unroll=2)
            def row_body(c):
                scale = jnp.where(row0 + c == b_last,
                                  jnp.float32(1.0), jnp.float32(inv_s))
                for g in range(n_groups):
                    sl = pl.ds(g * _LANES, _LANES)
                    acc = rows_v[buf, c, sl]
                    for s in range(1, n_streams - 1):
                        acc = acc + rows_v[buf, s * _C + c, sl]
                    out_v[buf, c, sl] = (
                        rows_v[buf, (n_streams - 1) * _C + c, sl] + scale * acc)

        idx_copy(0, 0).start()
        idx_copy(0, 0).wait()
        issue_gathers(0)
        idx_copy(1, 1).start()

        def pair_body(j, carry):
            for buf in (0, 1):
                jc = j + buf
                nbuf = 1 - buf

                @pl.when(jc + 1 < my_cpw)
                def _():
                    idx_copy(jc + 1, nbuf).wait()
                    issue_gathers(nbuf)

                drain_gathers(buf)

                @pl.when(jc + 2 < my_cpw)
                def _():
                    idx_copy(jc + 2, buf).start()

                @pl.when(jc >= 2)
                def _():
                    out_copy(jc - 2, buf).wait()

                compute(jc, buf)
                out_copy(jc, buf).start()
            return carry

        lax.fori_loop(0, my_cpw // 2, lambda i, cr: pair_body(2 * i, cr), 0)

        out_copy(my_cpw - 2, 0).wait()
        out_copy(my_cpw - 1, 1).wait()

    return sc_kernel


def kernel(nodes, neigh_idx, num_sample, table):
    del num_sample
    b, s = neigh_idx.shape
    _, d = table.shape

    n_min = -(-b // _C)
    cpw0 = _even_ceil(n_min * _CORE0_FRAC / _NUM_SUBCORES)
    rem = max(0, n_min - _NUM_SUBCORES * cpw0)
    cpw1 = max(2, _even_ceil(rem / _NUM_SUBCORES))
    n_chunks = _NUM_SUBCORES * (cpw0 + cpw1)
    b_pad = n_chunks * _C

    comb = jnp.concatenate(
        [neigh_idx.T.astype(jnp.int32), nodes[None, :].astype(jnp.int32)], axis=0)
    comb = jnp.pad(comb, ((0, 0), (0, b_pad - b)))
    idx_arr = (comb.reshape(s + 1, n_chunks, _C).transpose(1, 0, 2)
               .reshape(n_chunks * (s + 1) * _C))

    sc = _make_sc_kernel(cpw0, cpw1, s + 1, d, b - 1)
    out = sc(idx_arr, table)
    return out[:b]


# speedup vs baseline: 1.0807x; 1.0807x over previous
import functools

import jax
import jax.numpy as jnp
from jax import lax
from jax.experimental import pallas as pl
from jax.experimental.pallas import tpu as pltpu
from jax.experimental.pallas import tpu_sc as plsc

_C = 32
_NUM_CORES = 2
_NUM_SUBCORES = 16
_LANES = 16
_CORE0_FRAC = 0.84


def _even_ceil(x):
    v = int(-(-x // 1))
    return v + (v % 2)


def _make_sc_kernel(cpw0, cpw1, n_streams, d_feat, b_total):
    b_last = b_total - 1
    tail = b_total % _C
    inv_s = 1.0 / float(n_streams - 1)
    n_groups = d_feat // _LANES
    chunk_words = n_streams * _C
    mesh = plsc.VectorSubcoreMesh(
        core_axis_name="core", subcore_axis_name="subcore",
        num_cores=_NUM_CORES)

    @functools.partial(
        pl.kernel,
        out_type=jax.ShapeDtypeStruct((b_total, d_feat), jnp.float32),
        mesh=mesh,
        scratch_types=[
            pltpu.VMEM((2 * chunk_words,), jnp.int32),
            pltpu.VMEM((2, chunk_words, d_feat), jnp.float32),
            pltpu.VMEM((2, _C, d_feat), jnp.float32),
            pltpu.SemaphoreType.DMA,
            pltpu.SemaphoreType.DMA,
            pltpu.SemaphoreType.DMA,
            pltpu.SemaphoreType.DMA,
            pltpu.SemaphoreType.DMA,
            pltpu.SemaphoreType.DMA,
        ],
    )
    def sc_kernel(idx_hbm, table_hbm, out_hbm, idx_v, rows_v, out_v,
                  i_sem0, i_sem1, g_sem0, g_sem1, o_sem0, o_sem1):
        core = lax.axis_index("core")
        sub = lax.axis_index("subcore")
        my_cpw = jnp.where(core == 0, cpw0, cpw1)
        c0 = jnp.where(core == 0, sub * cpw0,
                       _NUM_SUBCORES * cpw0 + sub * cpw1)
        i_sems = (i_sem0, i_sem1)
        g_sems = (g_sem0, g_sem1)
        o_sems = (o_sem0, o_sem1)

        def idx_copy(j, buf):
            return pltpu.make_async_copy(
                idx_hbm.at[pl.ds((c0 + j) * chunk_words, chunk_words)],
                idx_v.at[pl.ds(buf * chunk_words, chunk_words)], i_sems[buf])

        def gather_copy(buf):
            return pltpu.make_async_copy(
                table_hbm.at[idx_v.at[pl.ds(buf * chunk_words, chunk_words)]],
                rows_v.at[buf], g_sems[buf])

        def issue_gathers(buf):
            gather_copy(buf).start()

        def drain_gathers(buf):
            gather_copy(buf).wait()

        def _out_do(j, buf, op):
            row0 = (c0 + j) * _C
            full = pltpu.make_async_copy(
                out_v.at[buf], out_hbm.at[pl.ds(row0, _C)], o_sems[buf])

            @pl.when(row0 + _C <= b_total)
            def _():
                getattr(full, op)()

            if tail:
                part = pltpu.make_async_copy(
                    out_v.at[buf, pl.ds(0, tail)],
                    out_hbm.at[pl.ds(row0, tail)], o_sems[buf])

                @pl.when((row0 < b_total) & (row0 + _C > b_total))
                def _():
                    getattr(part, op)()

        def out_start(j, buf):
            _out_do(j, buf, "start")

        def out_wait(j, buf):
            _out_do(j, buf, "wait")

        def compute(j, buf):
            row0 = (c0 + j) * _C

            def row_body(c, carry):
                scale = jnp.where(row0 + c == b_last,
                                  jnp.float32(1.0), jnp.float32(inv_s))
                for g in range(n_groups):
                    sl = pl.ds(g * _LANES, _LANES)
                    acc = rows_v[buf, c, sl]
                    for s in range(1, n_streams - 1):
                        acc = acc + rows_v[buf, s * _C + c, sl]
                    out_v[buf, c, sl] = (
                        rows_v[buf, (n_streams - 1) * _C + c, sl] + scale * acc)
                return carry

            lax.fori_loop(0, _C, row_body, 0)

        idx_copy(0, 0).start()
        idx_copy(0, 0).wait()
        issue_gathers(0)
        idx_copy(1, 1).start()

        def pair_body(j, carry):
            for buf in (0, 1):
                jc = j + buf
                nbuf = 1 - buf

                @pl.when(jc + 1 < my_cpw)
                def _():
                    idx_copy(jc + 1, nbuf).wait()
                    issue_gathers(nbuf)

                drain_gathers(buf)

                @pl.when(jc + 2 < my_cpw)
                def _():
                    idx_copy(jc + 2, buf).start()

                @pl.when(jc >= 2)
                def _():
                    out_wait(jc - 2, buf)

                compute(jc, buf)
                out_start(jc, buf)
            return carry

        lax.fori_loop(0, my_cpw // 2, lambda i, cr: pair_body(2 * i, cr), 0)

        out_wait(my_cpw - 2, 0)
        out_wait(my_cpw - 1, 1)

    return sc_kernel


def kernel(nodes, neigh_idx, num_sample, table):
    del num_sample
    b, s = neigh_idx.shape
    _, d = table.shape

    n_min = -(-b // _C)
    cpw0 = _even_ceil(n_min * _CORE0_FRAC / _NUM_SUBCORES)
    rem = max(0, n_min - _NUM_SUBCORES * cpw0)
    cpw1 = max(2, _even_ceil(rem / _NUM_SUBCORES))
    n_chunks = _NUM_SUBCORES * (cpw0 + cpw1)
    b_pad = n_chunks * _C

    comb = jnp.concatenate(
        [neigh_idx.T.astype(jnp.int32), nodes[None, :].astype(jnp.int32)], axis=0)
    comb = jnp.pad(comb, ((0, 0), (0, b_pad - b)))
    idx_arr = (comb.reshape(s + 1, n_chunks, _C).transpose(1, 0, 2)
               .reshape(n_chunks * (s + 1) * _C))

    sc = _make_sc_kernel(cpw0, cpw1, s + 1, d, b)
    return sc(idx_arr, table)
